# Initial kernel scaffold; baseline (speedup 1.0000x reference)
import jax
import jax.numpy as jnp
from jax.experimental import pallas as pl


def kernel(features, tables):
    raise NotImplementedError("write your pallas kernel here")


# speedup vs baseline: 1.1935x; 1.1935x over previous
import functools

import jax
import jax.numpy as jnp
from jax import lax
from jax.experimental import pallas as pl
from jax.experimental.pallas import tpu as pltpu
from jax.experimental.pallas import tpu_sc as plsc

NUM_FEATURES = 26
VOCAB = 100000
EMBED_DIM = 32
BATCH = 16384

NUM_CORES = 2
NUM_SUBCORES = 16
NW = NUM_CORES * NUM_SUBCORES
TOTAL_ROWS = BATCH * NUM_FEATURES
ROWS_PER_W = TOTAL_ROWS // NW
CHUNK = 512
NCHUNK = ROWS_PER_W // CHUNK
LANES = 16

_mesh = plsc.VectorSubcoreMesh(core_axis_name="c", subcore_axis_name="s")


@functools.partial(
    pl.kernel,
    mesh=_mesh,
    compiler_params=pltpu.CompilerParams(use_tc_tiling_on_sc=False),
    out_type=jax.ShapeDtypeStruct((TOTAL_ROWS, EMBED_DIM), jnp.float32),
    scratch_types=[
        pltpu.VMEM((ROWS_PER_W,), jnp.int32),
        pltpu.VMEM((ROWS_PER_W,), jnp.int32),
        pltpu.VMEM((CHUNK, EMBED_DIM), jnp.float32),
        pltpu.SemaphoreType.DMA,
    ],
)
def _embed(feat_h, offs_h, table_h, out_h, idx_v, offs_v, buf_v, sem):
    wid = lax.axis_index("s") * NUM_CORES + lax.axis_index("c")
    base = pl.multiple_of(wid * ROWS_PER_W, ROWS_PER_W)

    pltpu.sync_copy(feat_h.at[pl.ds(base, ROWS_PER_W)], idx_v)
    pltpu.sync_copy(offs_h, offs_v)

    def add_body(i, carry):
        s = pl.ds(i * LANES, LANES)
        idx_v[s] = idx_v[s] + offs_v[s]
        return carry

    lax.fori_loop(0, ROWS_PER_W // LANES, add_body, 0)

    def chunk_body(c, carry):
        off = pl.multiple_of(c * CHUNK, CHUNK)
        pltpu.async_copy(table_h.at[idx_v.at[pl.ds(off, CHUNK)]], buf_v, sem).wait()
        pltpu.sync_copy(buf_v, out_h.at[pl.ds(base + off, CHUNK)])
        return carry

    lax.fori_loop(0, NCHUNK, chunk_body, 0)


def kernel(features, tables):
    feat_flat = features.astype(jnp.int32).reshape(TOTAL_ROWS)
    table_flat = tables.reshape(NUM_FEATURES * VOCAB, EMBED_DIM)
    offs = jnp.tile(
        jnp.arange(NUM_FEATURES, dtype=jnp.int32) * VOCAB, ROWS_PER_W // NUM_FEATURES
    )
    out = _embed(feat_flat, offs, table_flat)
    return out.reshape(BATCH, NUM_FEATURES * EMBED_DIM)


# speedup vs baseline: 1.2105x; 1.0142x over previous
import functools

import jax
import jax.numpy as jnp
from jax import lax
from jax.experimental import pallas as pl
from jax.experimental.pallas import tpu as pltpu
from jax.experimental.pallas import tpu_sc as plsc

NUM_FEATURES = 26
VOCAB = 100000
EMBED_DIM = 32
BATCH = 16384

NUM_CORES = 2
NUM_SUBCORES = 16
NW = NUM_CORES * NUM_SUBCORES
TOTAL_ROWS = BATCH * NUM_FEATURES
ROWS_PER_W = TOTAL_ROWS // NW
CHUNK = 512
NCHUNK = ROWS_PER_W // CHUNK
NBUF = 6
PIPE = 4
LANES = 16

_mesh = plsc.VectorSubcoreMesh(core_axis_name="c", subcore_axis_name="s")


@functools.partial(
    pl.kernel,
    mesh=_mesh,
    compiler_params=pltpu.CompilerParams(use_tc_tiling_on_sc=False),
    out_type=jax.ShapeDtypeStruct((TOTAL_ROWS, EMBED_DIM), jnp.float32),
    scratch_types=[
        pltpu.VMEM((ROWS_PER_W,), jnp.int32),
        pltpu.VMEM((ROWS_PER_W,), jnp.int32),
        pltpu.VMEM((NBUF, CHUNK, EMBED_DIM), jnp.float32),
    ]
    + [pltpu.SemaphoreType.DMA] * (2 * NBUF),
)
def _embed(feat_h, offs_h, table_h, out_h, idx_v, offs_v, buf_v, *sems):
    gsems = sems[:NBUF]
    wsems = sems[NBUF:]
    wid = lax.axis_index("s") * NUM_CORES + lax.axis_index("c")
    base = pl.multiple_of(wid * ROWS_PER_W, ROWS_PER_W)

    pltpu.sync_copy(feat_h.at[pl.ds(base, ROWS_PER_W)], idx_v)
    pltpu.sync_copy(offs_h, offs_v)

    def add_body(i, carry):
        s = pl.ds(i * LANES, LANES)
        idx_v[s] = idx_v[s] + offs_v[s]
        return carry

    lax.fori_loop(0, ROWS_PER_W // LANES, add_body, 0)

    gather_h = [None] * NCHUNK
    write_h = [None] * NCHUNK

    def start_gather(g):
        s = g % NBUF
        gather_h[g] = pltpu.async_copy(
            table_h.at[idx_v.at[pl.ds(g * CHUNK, CHUNK)]], buf_v.at[s], gsems[s]
        )

    def start_write(g):
        s = g % NBUF
        write_h[g] = pltpu.async_copy(
            buf_v.at[s], out_h.at[pl.ds(base + g * CHUNK, CHUNK)], wsems[s]
        )

    for g in range(NCHUNK):
        if g >= NBUF:
            write_h[g - NBUF].wait()
        start_gather(g)
        if g >= PIPE:
            gather_h[g - PIPE].wait()
            start_write(g - PIPE)
    for g in range(NCHUNK - PIPE, NCHUNK):
        gather_h[g].wait()
        start_write(g)
    for g in range(NCHUNK - NBUF, NCHUNK):
        write_h[g].wait()


def kernel(features, tables):
    feat_flat = features.astype(jnp.int32).reshape(TOTAL_ROWS)
    table_flat = tables.reshape(NUM_FEATURES * VOCAB, EMBED_DIM)
    offs = jnp.tile(
        jnp.arange(NUM_FEATURES, dtype=jnp.int32) * VOCAB, ROWS_PER_W // NUM_FEATURES
    )
    out = _embed(feat_flat, offs, table_flat)
    return out.reshape(BATCH, NUM_FEATURES * EMBED_DIM)
